# Initial kernel scaffold; baseline (speedup 1.0000x reference)
import jax
import jax.numpy as jnp
from jax.experimental import pallas as pl


def kernel(inputs, edges, W1, b1, W2, b2):
    raise NotImplementedError("write your pallas kernel here")


# speedup vs baseline: 1.7516x; 1.7516x over previous
import functools

import jax
import jax.numpy as jnp
from jax import lax
from jax.experimental import pallas as pl
from jax.experimental.pallas import tpu as pltpu
from jax.experimental.pallas import tpu_sc as plsc

NN = 10000
NE = 160000
DI = 8710
DH = 1024
DO = 70

NCHUNK = DH // 128
MB = 1024
KB = 512
NK = -(-DI // KB)
RPT = 40
NROWSP = 32 * RPT
ROWS_PER_SC = NROWSP // 2
NNS = 10240
MGRID = -(-NN // MB)


def _make_deg():
    mesh = plsc.VectorSubcoreMesh(core_axis_name="c", subcore_axis_name="s",
                                  num_cores=2, num_subcores=16)

    @functools.partial(
        pl.kernel,
        out_type=jax.ShapeDtypeStruct((32, NNS), jnp.float32),
        mesh=mesh,
        scratch_types=[
            pltpu.VMEM((RPT, 128), jnp.int32),
            pltpu.VMEM((NNS,), jnp.float32),
        ],
        compiler_params=pltpu.CompilerParams(needs_layout_passes=False),
    )
    def deg_kernel(dst_hbm, degp_hbm, dstb, dloc):
        c = lax.axis_index("c")
        s = lax.axis_index("s")
        w = c * 16 + s
        pltpu.sync_copy(dst_hbm.at[pl.ds(w * RPT, RPT)], dstb)

        @pl.loop(0, NNS // 16)
        def _(i):
            dloc[pl.ds(i * 16, 16)] = jnp.zeros((16,), jnp.float32)

        ones = jnp.ones((16,), jnp.float32)

        @pl.loop(0, RPT)
        def _(r):
            for v in range(8):
                idx = dstb[r, pl.ds(v * 16, 16)]
                plsc.addupdate_scatter(dloc, [idx], ones)

        pltpu.sync_copy(dloc, degp_hbm.at[w])

    return deg_kernel


_lazy = {}


def _deg(dst2):
    if "deg" not in _lazy:
        _lazy["deg"] = _make_deg()
    return _lazy["deg"](dst2)


def _mm1_body(x_ref, w_ref, degp_ref, *out_refs):
    k = pl.program_id(1)
    x = x_ref[...]
    col = lax.broadcasted_iota(jnp.int32, x.shape, 1) + k * KB
    xb = jnp.where(col < DI, x, 0.0).astype(jnp.bfloat16)
    w = w_ref[...]
    for ci in range(NCHUNK):
        part = jnp.dot(xb, w[:, ci * 128:(ci + 1) * 128],
                       preferred_element_type=jnp.float32)

        @pl.when(k == 0)
        def _():
            out_refs[ci][...] = part

        @pl.when(k > 0)
        def _():
            out_refs[ci][...] += part

    @pl.when(k == NK - 1)
    def _():
        dinv = lax.rsqrt(jnp.sum(degp_ref[...], axis=0) + 1.0)[:, None]
        for ci in range(NCHUNK):
            out_refs[ci][...] *= dinv


_mm1 = pl.pallas_call(
    _mm1_body,
    grid=(MGRID, NK),
    in_specs=[
        pl.BlockSpec((MB, KB), lambda i, k: (i, k)),
        pl.BlockSpec((KB, DH), lambda i, k: (k, 0)),
        pl.BlockSpec((32, MB), lambda i, k: (0, i)),
    ],
    out_specs=[pl.BlockSpec((MB, 128), lambda i, k: (i, 0))] * NCHUNK,
    out_shape=[jax.ShapeDtypeStruct((NN, 128), jnp.float32)] * NCHUNK,
    compiler_params=pltpu.CompilerParams(
        dimension_semantics=("parallel", "arbitrary")),
)


def _make_prop(nchunk):
    mesh = plsc.VectorSubcoreMesh(core_axis_name="c", subcore_axis_name="s",
                                  num_cores=2, num_subcores=16)
    scratch = [
        pltpu.VMEM((RPT, 128), jnp.int32),
        pltpu.VMEM((RPT, 128), jnp.int32),
        pltpu.VMEM((128, 128), jnp.float32),
        pltpu.VMEM((128, 128), jnp.float32),
        pltpu.VMEM((32, 128), jnp.float32),
        pltpu.VMEM_SHARED((NNS, 128), jnp.float32),
        pltpu.SemaphoreType.DMA,
        pltpu.SemaphoreType.DMA,
    ]

    @functools.partial(
        pl.kernel,
        out_type=[jax.ShapeDtypeStruct((2, NNS, 128), jnp.float32)] * nchunk,
        mesh=mesh,
        scratch_types=scratch,
        compiler_params=pltpu.CompilerParams(needs_layout_passes=False),
    )
    def prop(*refs):
        src_hbm, dst_hbm = refs[0], refs[1]
        y_refs = refs[2:2 + nchunk]
        s_refs = refs[2 + nchunk:2 + 2 * nchunk]
        srcb, dstb, g0, g1, zbuf, slab, sem0, sem1 = refs[2 + 2 * nchunk:]
        c = lax.axis_index("c")
        s = lax.axis_index("s")
        base = c * ROWS_PER_SC + s * RPT
        pltpu.sync_copy(src_hbm.at[pl.ds(base, RPT)], srcb)
        pltpu.sync_copy(dst_hbm.at[pl.ds(base, RPT)], dstb)

        @pl.loop(0, 32)
        def _(r):
            for v in range(8):
                zbuf[r, pl.ds(v * 16, 16)] = jnp.zeros((16,), jnp.float32)

        bufs = (g0, g1)
        sems = (sem0, sem1)
        zrow = s * (NNS // 16)
        for j in range(nchunk):
            for t in range(NNS // 16 // 32):
                pltpu.sync_copy(zbuf, slab.at[pl.ds(zrow + t * 32, 32)])
            plsc.subcore_barrier()

            descs = [None, None]
            descs[0] = pltpu.async_copy(y_refs[j].at[srcb.at[0]], bufs[0],
                                        sems[0])
            for i in range(RPT):
                b = i & 1
                if i + 1 < RPT:
                    nb = (i + 1) & 1
                    descs[nb] = pltpu.async_copy(
                        y_refs[j].at[srcb.at[i + 1]], bufs[nb], sems[nb])
                descs[b].wait()
                pltpu.sync_copy(bufs[b], slab.at[dstb.at[i]], add=True)

            plsc.subcore_barrier()
            pltpu.sync_copy(slab.at[pl.ds(zrow, NNS // 16)],
                            s_refs[j].at[c, pl.ds(zrow, NNS // 16)])
            plsc.subcore_barrier()

    return prop


def _prop(nchunk, *args):
    if nchunk not in _lazy:
        _lazy[nchunk] = _make_prop(nchunk)
    return _lazy[nchunk](*args)


def _mm2_body(degp_ref, w2_ref, b1_ref, *refs):
    s_refs = refs[:NCHUNK]
    y_refs = refs[NCHUNK:2 * NCHUNK]
    out_ref = refs[2 * NCHUNK]
    dinv = lax.rsqrt(jnp.sum(degp_ref[...], axis=0) + 1.0)[:, None]
    w2 = w2_ref[...]
    b1 = b1_ref[...]
    acc = jnp.zeros((MB, 128), jnp.float32)
    for ci in range(NCHUNK):
        sc0 = s_refs[ci][0]
        sc1 = s_refs[ci][1]
        h = dinv * (sc0 + sc1 + y_refs[ci][...]) \
            + b1[:, ci * 128:(ci + 1) * 128]
        h = jnp.maximum(h, 0.0).astype(jnp.bfloat16)
        acc = acc + jnp.dot(h, w2[ci * 128:(ci + 1) * 128, :],
                            preferred_element_type=jnp.float32)
    out_ref[...] = dinv * acc


_mm2 = pl.pallas_call(
    _mm2_body,
    grid=(MGRID,),
    in_specs=[
        pl.BlockSpec((32, MB), lambda i: (0, i)),
        pl.BlockSpec((DH, 128), lambda i: (0, 0)),
        pl.BlockSpec((1, DH), lambda i: (0, 0)),
    ] + [pl.BlockSpec((2, MB, 128), lambda i: (0, i, 0))] * NCHUNK
      + [pl.BlockSpec((MB, 128), lambda i: (i, 0))] * NCHUNK,
    out_specs=pl.BlockSpec((MB, 128), lambda i: (i, 0)),
    out_shape=jax.ShapeDtypeStruct((NN, 128), jnp.float32),
    compiler_params=pltpu.CompilerParams(dimension_semantics=("parallel",)),
)


def _final_body(degp_ref, b2_ref, s2_ref, y2_ref, out_ref):
    dinv = lax.rsqrt(jnp.sum(degp_ref[...], axis=0) + 1.0)[:, None]
    z = dinv * (s2_ref[0] + s2_ref[1] + y2_ref[...]) + b2_ref[...]
    col = lax.broadcasted_iota(jnp.int32, z.shape, 1)
    mask = col < DO
    zm = jnp.where(mask, z, -jnp.inf)
    m = jnp.max(zm, axis=1, keepdims=True)
    e = jnp.where(mask, jnp.exp(z - m), 0.0)
    ls = m + jnp.log(jnp.sum(e, axis=1, keepdims=True))
    out_ref[...] = (z - ls)[:, :DO]


_final = pl.pallas_call(
    _final_body,
    grid=(MGRID,),
    in_specs=[
        pl.BlockSpec((32, MB), lambda i: (0, i)),
        pl.BlockSpec((1, 128), lambda i: (0, 0)),
        pl.BlockSpec((2, MB, 128), lambda i: (0, i, 0)),
        pl.BlockSpec((MB, 128), lambda i: (i, 0)),
    ],
    out_specs=pl.BlockSpec((MB, DO), lambda i: (i, 0)),
    out_shape=jax.ShapeDtypeStruct((NN, DO), jnp.float32),
    compiler_params=pltpu.CompilerParams(dimension_semantics=("parallel",)),
)


def kernel(inputs, edges, W1, b1, W2, b2):
    edges = edges.astype(jnp.int32)
    npad = NROWSP * 128 - NE
    src2 = jnp.concatenate(
        [edges[0], jnp.zeros((npad,), jnp.int32)]).reshape(NROWSP, 128)
    dst2 = jnp.concatenate(
        [edges[1], jnp.full((npad,), NN, jnp.int32)]).reshape(NROWSP, 128)
    w1p = jnp.pad(W1, ((0, NK * KB - DI), (0, 0))).astype(jnp.bfloat16)
    w2p = jnp.pad(W2, ((0, 0), (0, 128 - DO))).astype(jnp.bfloat16)
    b1r = b1.reshape(1, DH)
    b2r = jnp.pad(b2, (0, 128 - DO)).reshape(1, 128)

    degp = _deg(dst2)
    y1 = _mm1(inputs, w1p, degp)
    s1 = _prop(NCHUNK, src2, dst2, *y1)
    y2 = _mm2(degp, w2p, b1r, *s1, *y1)
    s2 = _prop(1, src2, dst2, y2)
    if isinstance(s2, (list, tuple)):
        s2 = s2[0]
    return _final(degp, b2r, s2, y2)


# speedup vs baseline: 2.2674x; 1.2945x over previous
import functools

import jax
import jax.numpy as jnp
from jax import lax
from jax.experimental import pallas as pl
from jax.experimental.pallas import tpu as pltpu
from jax.experimental.pallas import tpu_sc as plsc

NN = 10000
NE = 160000
DI = 8710
DH = 1024
DO = 70

NCHUNK = DH // 128
MB = 1024
KB = 2048
NK = -(-DI // KB)
RPT = 40
NROWSP = 32 * RPT
ROWS_PER_SC = NROWSP // 2
NNS = 10240
MGRID = -(-NN // MB)


def _make_deg():
    mesh = plsc.VectorSubcoreMesh(core_axis_name="c", subcore_axis_name="s",
                                  num_cores=2, num_subcores=16)

    @functools.partial(
        pl.kernel,
        out_type=jax.ShapeDtypeStruct((32, NNS), jnp.float32),
        mesh=mesh,
        scratch_types=[
            pltpu.VMEM((RPT, 128), jnp.int32),
            pltpu.VMEM((NNS,), jnp.float32),
        ],
        compiler_params=pltpu.CompilerParams(needs_layout_passes=False),
    )
    def deg_kernel(dst_hbm, degp_hbm, dstb, dloc):
        c = lax.axis_index("c")
        s = lax.axis_index("s")
        w = c * 16 + s
        pltpu.sync_copy(dst_hbm.at[pl.ds(w * RPT, RPT)], dstb)

        @pl.loop(0, NNS // 16)
        def _(i):
            dloc[pl.ds(i * 16, 16)] = jnp.zeros((16,), jnp.float32)

        ones = jnp.ones((16,), jnp.float32)

        @pl.loop(0, RPT)
        def _(r):
            for v in range(8):
                idx = dstb[r, pl.ds(v * 16, 16)]
                plsc.addupdate_scatter(dloc, [idx], ones)

        pltpu.sync_copy(dloc, degp_hbm.at[w])

    return deg_kernel


_lazy = {}


def _deg(dst2):
    if "deg" not in _lazy:
        _lazy["deg"] = _make_deg()
    return _lazy["deg"](dst2)


def _mm1_body(x_ref, w_ref, degp_ref, *refs):
    out_refs = refs[:NCHUNK]
    acc_ref = refs[NCHUNK]
    k = pl.program_id(1)
    x = x_ref[...]
    col = lax.broadcasted_iota(jnp.int32, x.shape, 1) + k * KB
    xb = jnp.where(col < DI, x, 0.0).astype(jnp.bfloat16)
    part = jnp.dot(xb, w_ref[...], preferred_element_type=jnp.float32)

    @pl.when(k == 0)
    def _():
        acc_ref[...] = part

    @pl.when(k > 0)
    def _():
        acc_ref[...] += part

    @pl.when(k == NK - 1)
    def _():
        dinv = lax.rsqrt(jnp.sum(degp_ref[...], axis=0) + 1.0)[:, None]
        acc = acc_ref[...]
        for ci in range(NCHUNK):
            out_refs[ci][...] = dinv * acc[:, ci * 128:(ci + 1) * 128]


_mm1 = pl.pallas_call(
    _mm1_body,
    grid=(MGRID, NK),
    in_specs=[
        pl.BlockSpec((MB, KB), lambda i, k: (i, k)),
        pl.BlockSpec((KB, DH), lambda i, k: (k, 0)),
        pl.BlockSpec((32, MB), lambda i, k: (0, i)),
    ],
    out_specs=[pl.BlockSpec((MB, 128), lambda i, k: (i, 0))] * NCHUNK,
    out_shape=[jax.ShapeDtypeStruct((NN, 128), jnp.float32)] * NCHUNK,
    scratch_shapes=[pltpu.VMEM((MB, DH), jnp.float32)],
    compiler_params=pltpu.CompilerParams(
        dimension_semantics=("parallel", "arbitrary")),
)


def _make_prop(nchunk):
    mesh = plsc.VectorSubcoreMesh(core_axis_name="c", subcore_axis_name="s",
                                  num_cores=2, num_subcores=16)
    scratch = [
        pltpu.VMEM((RPT, 128), jnp.int32),
        pltpu.VMEM((RPT, 128), jnp.int32),
        pltpu.VMEM((128, 128), jnp.float32),
        pltpu.VMEM((128, 128), jnp.float32),
        pltpu.VMEM((32, 128), jnp.float32),
        pltpu.VMEM_SHARED((NNS, 128), jnp.float32),
        pltpu.SemaphoreType.DMA,
        pltpu.SemaphoreType.DMA,
    ]

    @functools.partial(
        pl.kernel,
        out_type=[jax.ShapeDtypeStruct((2, NNS, 128), jnp.float32)] * nchunk,
        mesh=mesh,
        scratch_types=scratch,
        compiler_params=pltpu.CompilerParams(needs_layout_passes=False),
    )
    def prop(*refs):
        src_hbm, dst_hbm = refs[0], refs[1]
        y_refs = refs[2:2 + nchunk]
        s_refs = refs[2 + nchunk:2 + 2 * nchunk]
        srcb, dstb, g0, g1, zbuf, slab, sem0, sem1 = refs[2 + 2 * nchunk:]
        c = lax.axis_index("c")
        s = lax.axis_index("s")
        base = c * ROWS_PER_SC + s * RPT
        pltpu.sync_copy(src_hbm.at[pl.ds(base, RPT)], srcb)
        pltpu.sync_copy(dst_hbm.at[pl.ds(base, RPT)], dstb)

        @pl.loop(0, 32)
        def _(r):
            for v in range(8):
                zbuf[r, pl.ds(v * 16, 16)] = jnp.zeros((16,), jnp.float32)

        bufs = (g0, g1)
        sems = (sem0, sem1)
        zrow = s * (NNS // 16)
        for j in range(nchunk):
            for t in range(NNS // 16 // 32):
                pltpu.sync_copy(zbuf, slab.at[pl.ds(zrow + t * 32, 32)])
            plsc.subcore_barrier()

            descs = [None, None]
            descs[0] = pltpu.async_copy(y_refs[j].at[srcb.at[0]], bufs[0],
                                        sems[0])
            for i in range(RPT):
                b = i & 1
                if i + 1 < RPT:
                    nb = (i + 1) & 1
                    descs[nb] = pltpu.async_copy(
                        y_refs[j].at[srcb.at[i + 1]], bufs[nb], sems[nb])
                descs[b].wait()
                pltpu.sync_copy(bufs[b], slab.at[dstb.at[i]], add=True)

            plsc.subcore_barrier()
            pltpu.sync_copy(slab.at[pl.ds(zrow, NNS // 16)],
                            s_refs[j].at[c, pl.ds(zrow, NNS // 16)])
            plsc.subcore_barrier()

    return prop


def _prop(nchunk, *args):
    if nchunk not in _lazy:
        _lazy[nchunk] = _make_prop(nchunk)
    return _lazy[nchunk](*args)


def _mm2_body(degp_ref, w2_ref, b1_ref, *refs):
    s_refs = refs[:NCHUNK]
    y_refs = refs[NCHUNK:2 * NCHUNK]
    out_ref = refs[2 * NCHUNK]
    dinv = lax.rsqrt(jnp.sum(degp_ref[...], axis=0) + 1.0)[:, None]
    w2 = w2_ref[...]
    b1 = b1_ref[...]
    acc = jnp.zeros((MB, 128), jnp.float32)
    for ci in range(NCHUNK):
        sc0 = s_refs[ci][0]
        sc1 = s_refs[ci][1]
        h = dinv * (sc0 + sc1 + y_refs[ci][...]) \
            + b1[:, ci * 128:(ci + 1) * 128]
        h = jnp.maximum(h, 0.0).astype(jnp.bfloat16)
        acc = acc + jnp.dot(h, w2[ci * 128:(ci + 1) * 128, :],
                            preferred_element_type=jnp.float32)
    out_ref[...] = dinv * acc


_mm2 = pl.pallas_call(
    _mm2_body,
    grid=(MGRID,),
    in_specs=[
        pl.BlockSpec((32, MB), lambda i: (0, i)),
        pl.BlockSpec((DH, 128), lambda i: (0, 0)),
        pl.BlockSpec((1, DH), lambda i: (0, 0)),
    ] + [pl.BlockSpec((2, MB, 128), lambda i: (0, i, 0))] * NCHUNK
      + [pl.BlockSpec((MB, 128), lambda i: (i, 0))] * NCHUNK,
    out_specs=pl.BlockSpec((MB, 128), lambda i: (i, 0)),
    out_shape=jax.ShapeDtypeStruct((NN, 128), jnp.float32),
    compiler_params=pltpu.CompilerParams(dimension_semantics=("parallel",)),
)


def _final_body(degp_ref, b2_ref, s2_ref, y2_ref, out_ref):
    dinv = lax.rsqrt(jnp.sum(degp_ref[...], axis=0) + 1.0)[:, None]
    z = dinv * (s2_ref[0] + s2_ref[1] + y2_ref[...]) + b2_ref[...]
    col = lax.broadcasted_iota(jnp.int32, z.shape, 1)
    mask = col < DO
    zm = jnp.where(mask, z, -jnp.inf)
    m = jnp.max(zm, axis=1, keepdims=True)
    e = jnp.where(mask, jnp.exp(z - m), 0.0)
    ls = m + jnp.log(jnp.sum(e, axis=1, keepdims=True))
    out_ref[...] = (z - ls)[:, :DO]


_final = pl.pallas_call(
    _final_body,
    grid=(MGRID,),
    in_specs=[
        pl.BlockSpec((32, MB), lambda i: (0, i)),
        pl.BlockSpec((1, 128), lambda i: (0, 0)),
        pl.BlockSpec((2, MB, 128), lambda i: (0, i, 0)),
        pl.BlockSpec((MB, 128), lambda i: (i, 0)),
    ],
    out_specs=pl.BlockSpec((MB, DO), lambda i: (i, 0)),
    out_shape=jax.ShapeDtypeStruct((NN, DO), jnp.float32),
    compiler_params=pltpu.CompilerParams(dimension_semantics=("parallel",)),
)


def kernel(inputs, edges, W1, b1, W2, b2):
    edges = edges.astype(jnp.int32)
    npad = NROWSP * 128 - NE
    src2 = jnp.concatenate(
        [edges[0], jnp.zeros((npad,), jnp.int32)]).reshape(NROWSP, 128)
    dst2 = jnp.concatenate(
        [edges[1], jnp.full((npad,), NN, jnp.int32)]).reshape(NROWSP, 128)
    w1p = jnp.pad(W1, ((0, NK * KB - DI), (0, 0))).astype(jnp.bfloat16)
    w2p = jnp.pad(W2, ((0, 0), (0, 128 - DO))).astype(jnp.bfloat16)
    b1r = b1.reshape(1, DH)
    b2r = jnp.pad(b2, (0, 128 - DO)).reshape(1, 128)

    degp = _deg(dst2)
    y1 = _mm1(inputs, w1p, degp)
    s1 = _prop(NCHUNK, src2, dst2, *y1)
    y2 = _mm2(degp, w2p, b1r, *s1, *y1)
    s2 = _prop(1, src2, dst2, y2)
    if isinstance(s2, (list, tuple)):
        s2 = s2[0]
    return _final(degp, b2r, s2, y2)
